# Initial kernel scaffold; baseline (speedup 1.0000x reference)
import jax
import jax.numpy as jnp
from jax.experimental import pallas as pl


def kernel(grid_th, guide_th, image_th, W_conv, b_conv):
    raise NotImplementedError("write your pallas kernel here")


# speedup vs baseline: 389.4150x; 389.4150x over previous
import jax
import jax.numpy as jnp
from jax.experimental import pallas as pl
from jax.experimental.pallas import tpu as pltpu

_B, _H, _W, _CIN = 4, 512, 512, 3
_GH, _GW, _GD, _GC = 16, 16, 8, 12
_COUT = _GC // (_CIN + 1)
_HC = 16


def _tc_body(grid_ref, guide_ref, img_ref, w_ref, b_ref, out_ref):
    f32 = jnp.float32

    conv = []
    padded = []
    for ci in range(_CIN):
        ich = img_ref[0, ci]
        hp = jnp.concatenate(
            [jnp.zeros((_H, 128), f32), ich, jnp.zeros((_H, 128), f32)], axis=1)
        vp = jnp.concatenate(
            [jnp.zeros((8, _W + 256), f32), hp, jnp.zeros((8, _W + 256), f32)],
            axis=0)
        padded.append(vp)
    for co in range(_CIN):
        acc = jnp.full((_H, _W), b_ref[co], f32)
        for dy in range(3):
            for dx in range(3):
                for ci in range(_CIN):
                    w = w_ref[dy, dx, ci, co]
                    acc = acc + w * jax.lax.slice(
                        padded[ci], (8 + dy - 1, 128 + dx - 1),
                        (8 + dy - 1 + _H, 128 + dx - 1 + _W))
        conv.append(acc)

    colf = jax.lax.broadcasted_iota(jnp.int32, (1, _W), 1).astype(f32)
    gx = (colf + 0.5) * (_GW / _W)
    wx1_full = gx - 0.5 - jnp.floor(gx - 0.5)
    rowf = jax.lax.broadcasted_iota(jnp.int32, (_H, 1), 0).astype(f32)
    gy = (rowf + 0.5) * (_GH / _H)
    wy1_full = gy - 0.5 - jnp.floor(gy - 0.5)

    segs = []
    for wb in range(_W // 16):
        fx = (wb - 1) // 2
        x0 = min(max(fx, 0), _GW - 1)
        x1 = min(max(fx + 1, 0), _GW - 1)
        wx1 = jax.lax.slice(wx1_full, (0, 16 * wb),
                            (1, 16 * wb + 16)).reshape(1, 1, 1, 16)
        g0 = grid_ref[0, :, x0]
        g1 = grid_ref[0, :, x1]
        seg = g0[..., None] * (1.0 - wx1) + g1[..., None] * wx1
        segs.append(seg)
    R = jnp.concatenate(segs, axis=-1)

    for hc in range(_H // _HC):
        fy = (hc - 1) // 2
        y0 = min(max(fy, 0), _GH - 1)
        y1 = min(max(fy + 1, 0), _GH - 1)
        wy1 = jax.lax.slice(wy1_full, (_HC * hc, 0),
                            (_HC * hc + _HC, 1)).reshape(1, 1, _HC, 1)
        Ry0 = jax.lax.slice(R, (y0, 0, 0, 0), (y0 + 1, _GD, _GC, _W))
        Ry1 = jax.lax.slice(R, (y1, 0, 0, 0), (y1 + 1, _GD, _GC, _W))
        P = (Ry0.reshape(_GD, _GC, 1, _W) * (1.0 - wy1)
             + Ry1.reshape(_GD, _GC, 1, _W) * wy1)

        rows = pl.ds(_HC * hc, _HC)
        gch = guide_ref[0, rows, :]
        gz = jnp.clip(gch, 0.0, 1.0) * _GD
        fz = jnp.floor(gz - 0.5)
        wz1 = gz - 0.5 - fz
        z0 = jnp.clip(fz, 0.0, _GD - 1.0)
        z1 = jnp.clip(fz + 1.0, 0.0, _GD - 1.0)
        wzs = []
        for z in range(_GD):
            zf = float(z)
            m = (jnp.where(z0 == zf, 1.0 - wz1, 0.0)
                 + jnp.where(z1 == zf, wz1, 0.0))
            wzs.append(m)
        WZ = jnp.stack(wzs)

        T = P * WZ[:, None, :, :]
        coeff = jnp.sum(T, axis=0)

        for co in range(_COUT):
            res = coeff[(_CIN + 1) * co + _CIN]
            for ci in range(_CIN):
                ich = jax.lax.slice(conv[ci], (_HC * hc, 0),
                                    (_HC * hc + _HC, _W))
                res = res + coeff[(_CIN + 1) * co + ci] * ich
            out_ref[0, co, rows, :] = res


def _run_tc(grid_th, guide_th, image_t, W_conv, b_conv):
    return pl.pallas_call(
        _tc_body,
        grid=(_B,),
        in_specs=[
            pl.BlockSpec((1, _GH, _GW, _GD, _GC), lambda b: (b, 0, 0, 0, 0)),
            pl.BlockSpec((1, _H, _W), lambda b: (b, 0, 0)),
            pl.BlockSpec((1, _CIN, _H, _W), lambda b: (b, 0, 0, 0)),
            pl.BlockSpec(memory_space=pltpu.SMEM),
            pl.BlockSpec(memory_space=pltpu.SMEM),
        ],
        out_specs=pl.BlockSpec((1, _COUT, _H, _W), lambda b: (b, 0, 0, 0)),
        out_shape=jax.ShapeDtypeStruct((_B, _COUT, _H, _W), jnp.float32),
    )(grid_th, guide_th, image_t, W_conv, b_conv)


def kernel(grid_th, guide_th, image_th, W_conv, b_conv):
    image_t = jnp.transpose(image_th, (0, 3, 1, 2))
    out_t = _run_tc(grid_th, guide_th, image_t, W_conv, b_conv)
    return jnp.transpose(out_t, (0, 2, 3, 1))
